# Initial kernel scaffold; baseline (speedup 1.0000x reference)
import jax
import jax.numpy as jnp
from jax.experimental import pallas as pl


def kernel(pred_cls_logits, pred_box_deltas, proposal_boxes, origins):
    raise NotImplementedError("write your pallas kernel here")


# speedup vs baseline: 1.8598x; 1.8598x over previous
import functools

import jax
import jax.numpy as jnp
from jax.experimental import pallas as pl
from jax.experimental.pallas import tpu as pltpu

_B = 4
_N = 20000
_C = 18
_PRE = 2048
_POST = 256
_IOU_THR = 0.25
_SCORE_THR = 0.05

_T = 256
_NT = _PRE // _T


def _nms_body(br_ref, bc_ref, out_ref, ovtt_ref):
    f32 = jnp.float32
    br = br_ref[0]
    bc = bc_ref[0]

    vol_r = jnp.ones((_PRE, 1), f32)
    vol_c = jnp.ones((1, _PRE), f32)
    for d in range(3):
        vol_r = vol_r * jnp.maximum(br[:, 3 + d:4 + d] - br[:, d:d + 1], 0.0)
        vol_c = vol_c * jnp.maximum(bc[3 + d:4 + d, :] - bc[d:d + 1, :], 0.0)

    def ov_block(p, t):
        i0, j0 = p * _T, t * _T
        inter = jnp.ones((_T, _T), f32)
        for d in range(3):
            lo = jnp.maximum(br[i0:i0 + _T, d:d + 1], bc[d:d + 1, j0:j0 + _T])
            hi = jnp.minimum(br[i0:i0 + _T, 3 + d:4 + d],
                             bc[3 + d:4 + d, j0:j0 + _T])
            inter = inter * jnp.maximum(hi - lo, 0.0)
        union = vol_r[i0:i0 + _T, :] + vol_c[:, j0:j0 + _T] - inter
        ovl = inter > _IOU_THR * jnp.maximum(union, 1e-9)
        same = br[i0:i0 + _T, 7:8] == bc[7:8, j0:j0 + _T]
        return (ovl & same).astype(f32)

    iota_l = jax.lax.broadcasted_iota(jnp.int32, (1, _T), 1)
    valid = (bc[6:7, :] > 0.0).astype(f32)

    keeps = []
    for t in range(_NT):
        alive = valid[:, t * _T:(t + 1) * _T]
        if t > 0:
            sup = jnp.zeros((1, _T), f32)
            for p in range(t):
                sup = sup + jax.lax.dot(keeps[p], ov_block(p, t),
                                        preferred_element_type=f32)
            alive = alive * (sup <= 0.0).astype(f32)
        ovtt_ref[...] = ov_block(t, t)

        def body(i, alive):
            onehot = iota_l == i
            ki = jnp.max(jnp.where(onehot, alive, 0.0))
            row = ovtt_ref[pl.ds(i, 1), :]
            gt = (iota_l > i).astype(f32)
            return alive * (1.0 - ki * row * gt)

        alive = jax.lax.fori_loop(0, _T, body, alive)
        keeps.append(alive)

    keepflat = jnp.concatenate(keeps, axis=1)
    total_kept = jnp.sum(keepflat)

    ii = jax.lax.broadcasted_iota(jnp.int32, (128, 128), 0)
    jj = jax.lax.broadcasted_iota(jnp.int32, (128, 128), 1)
    lt = (ii <= jj).astype(f32)
    r256 = jax.lax.broadcasted_iota(jnp.int32, (_POST, 1), 0).astype(f32)

    acc = jnp.zeros((_POST, 8), f32)
    kept_off = 0.0
    nk_off = total_kept
    for c in range(_PRE // 128):
        kc = keepflat[:, c * 128:(c + 1) * 128]
        nkc = 1.0 - kc
        kpre = jax.lax.dot(kc, lt, preferred_element_type=f32)
        nkpre = jax.lax.dot(nkc, lt, preferred_element_type=f32)
        pos = jnp.where(kc > 0.0, kept_off + kpre - 1.0, nk_off + nkpre - 1.0)
        perm = (r256 == pos).astype(f32)
        acc = acc + jax.lax.dot(perm, br[c * 128:(c + 1) * 128, :],
                                preferred_element_type=f32)
        kept_off = kept_off + jnp.sum(kc)
        nk_off = nk_off + jnp.sum(nkc)

    lane8 = jax.lax.broadcasted_iota(jnp.int32, (_POST, 8), 1)
    score_fixed = jnp.where(r256 < total_kept, acc[:, 6:7], -1.0)
    out_ref[0] = jnp.where(lane8 == 6, jnp.broadcast_to(score_fixed,
                                                        (_POST, 8)), acc)


@jax.jit
def kernel(pred_cls_logits, pred_box_deltas, proposal_boxes, origins):
    f32 = jnp.float32
    scores = jax.nn.softmax(pred_cls_logits, axis=-1)[..., :-1]
    cand = jnp.where(scores > _SCORE_THR, scores, -1.0).reshape(_B, _N * _C)
    top_s, top_i = jax.lax.top_k(cand, _PRE)

    cls = top_i % _C
    delt = jnp.take_along_axis(pred_box_deltas.reshape(_B, _N * _C, 6),
                               top_i[..., None], axis=1)
    n_idx = top_i // _C
    prop = jnp.take_along_axis(proposal_boxes, n_idx[..., None], axis=1)
    orig = jnp.take_along_axis(origins, n_idx[..., None], axis=1)
    box = prop + delt
    lo = orig - box[..., :3]
    hi = orig + box[..., 3:]
    br = jnp.concatenate([lo, hi, top_s[..., None],
                          cls.astype(f32)[..., None]], axis=-1)
    bc = jnp.swapaxes(br, 1, 2)

    outk = pl.pallas_call(
        _nms_body,
        grid=(_B,),
        in_specs=[
            pl.BlockSpec((1, _PRE, 8), lambda b: (b, 0, 0)),
            pl.BlockSpec((1, 8, _PRE), lambda b: (b, 0, 0)),
        ],
        out_specs=pl.BlockSpec((1, _POST, 8), lambda b: (b, 0, 0)),
        out_shape=jax.ShapeDtypeStruct((_B, _POST, 8), f32),
        scratch_shapes=[pltpu.VMEM((_T, _T), f32)],
    )(br, bc)

    return outk[..., :7], outk[..., 7].astype(jnp.int32)
